# baseline (speedup 1.0000x reference)
import functools

import jax
import jax.numpy as jnp
from jax import lax
from jax.experimental import pallas as pl
from jax.experimental.pallas import tpu as pltpu
from jax.experimental.pallas import tpu_sc as plsc

_NUM_WORKERS = 32
_BLK = 1024


def _embed_lookup_planes(table_t, seq_t, v, d, h, b):
    n_bb = b // _BLK
    n_tasks = (h * n_bb) // _NUM_WORKERS
    mesh = plsc.VectorSubcoreMesh(core_axis_name="c", subcore_axis_name="s")

    @functools.partial(
        pl.kernel,
        mesh=mesh,
        out_type=jax.ShapeDtypeStruct((h * d, b), jnp.float32),
        scratch_types=[
            pltpu.VMEM((_BLK,), jnp.int32),
            pltpu.VMEM((d, _BLK), jnp.float32),
            pltpu.SemaphoreType.DMA,
        ],
        compiler_params=pltpu.CompilerParams(use_tc_tiling_on_sc=False),
    )
    def k(table_hbm, seq_hbm, out_hbm, idx_v, slab_v, sem):
        wid = lax.axis_index("s") * 2 + lax.axis_index("c")

        def task(t, carry):
            tid = wid * n_tasks + t
            hh = tid // n_bb
            bb = (tid % n_bb) * _BLK
            pltpu.sync_copy(seq_hbm.at[hh, pl.ds(bb, _BLK)], idx_v)
            for e in range(d):
                pltpu.make_async_copy(
                    table_hbm.at[e].at[idx_v], slab_v.at[e], sem).start()
            for e in range(d):
                pltpu.make_async_copy(
                    table_hbm.at[e].at[idx_v], slab_v.at[e], sem).wait()
            pltpu.sync_copy(
                slab_v, out_hbm.at[pl.ds(hh * d, d), pl.ds(bb, _BLK)])
            return carry

        lax.fori_loop(0, n_tasks, task, 0)

    return k(table_t, seq_t)


def kernel(seq, table):
    b, h = seq.shape
    v, d = table.shape
    table_t = table.T
    seq_t = seq.T.astype(jnp.int32)
    out = _embed_lookup_planes(table_t, seq_t, v, d, h, b)
    return out.reshape(h, d, b).transpose(2, 0, 1)


# speedup vs baseline: 2.4895x; 2.4895x over previous
import functools

import jax
import jax.numpy as jnp
from jax import lax
from jax.experimental import pallas as pl
from jax.experimental.pallas import tpu as pltpu
from jax.experimental.pallas import tpu_sc as plsc

_NUM_WORKERS = 32
_BLK = 1024
_LANES = 16


def _embed_lookup(table, seq_t, v, d, h, b):
    n_bb = b // _BLK
    n_tasks = (h * n_bb) // _NUM_WORKERS
    blocks_per_slab = _BLK // 128
    mesh = plsc.VectorSubcoreMesh(core_axis_name="c", subcore_axis_name="s")

    @functools.partial(
        pl.kernel,
        mesh=mesh,
        out_type=jax.ShapeDtypeStruct((h * (d // 8) * (b // 128), 8, 128),
                                      jnp.float32),
        scratch_types=[
            pltpu.VMEM((_BLK,), jnp.int32),
            pltpu.VMEM((_BLK, d), jnp.float32),
            pltpu.VMEM((d, _BLK), jnp.float32),
            pltpu.SemaphoreType.DMA,
            pltpu.SemaphoreType.DMA,
        ],
        compiler_params=pltpu.CompilerParams(use_tc_tiling_on_sc=False,
                                             needs_layout_passes=False),
    )
    def k(table_hbm, seq_hbm, out_hbm, idx_v, rows_v, slab_v, gsem, osem):
        wid = lax.axis_index("s") * 2 + lax.axis_index("c")
        lane_iota = lax.iota(jnp.int32, _LANES)

        def task(t, carry):
            tid = wid * n_tasks + t
            hh = tid // n_bb
            bb = tid % n_bb
            pltpu.sync_copy(seq_hbm.at[hh, pl.ds(bb * _BLK, _BLK)], idx_v)
            pltpu.make_async_copy(table_hbm.at[idx_v], rows_v, gsem).start()
            pltpu.make_async_copy(table_hbm.at[idx_v], rows_v, gsem).wait()
            for j0 in range(0, _BLK, _LANES):
                row_ids = j0 + lane_iota
                for e in range(d):
                    col = plsc.load_gather(
                        rows_v, [row_ids, jnp.full((_LANES,), e, jnp.int32)])
                    slab_v[e, pl.ds(j0, _LANES)] = col
            for tr in range(d // 8):
                for j in range(blocks_per_slab):
                    m = (hh * (d // 8) + tr) * (b // 128) \
                        + bb * blocks_per_slab + j
                    pltpu.make_async_copy(
                        slab_v.at[pl.ds(tr * 8, 8), pl.ds(j * 128, 128)],
                        out_hbm.at[m], osem).start()
            for tr in range(d // 8):
                for j in range(blocks_per_slab):
                    m = (hh * (d // 8) + tr) * (b // 128) \
                        + bb * blocks_per_slab + j
                    pltpu.make_async_copy(
                        slab_v.at[pl.ds(tr * 8, 8), pl.ds(j * 128, 128)],
                        out_hbm.at[m], osem).wait()
            return carry

        lax.fori_loop(0, n_tasks, task, 0)

    return k(table, seq_t)


def kernel(seq, table):
    b, h = seq.shape
    v, d = table.shape
    seq_t = seq.T.astype(jnp.int32)
    out = _embed_lookup(table, seq_t, v, d, h, b)
    out5 = out.reshape(h, d // 8, b // 128, 8, 128)
    return out5.transpose(2, 4, 0, 1, 3).reshape(b, h, d)
